# Initial kernel scaffold; baseline (speedup 1.0000x reference)
import jax
import jax.numpy as jnp
from jax.experimental import pallas as pl


def kernel(x, pe):
    raise NotImplementedError("write your pallas kernel here")


# speedup vs baseline: 3.8818x; 3.8818x over previous
import functools

import jax
import jax.numpy as jnp
from jax import lax
from jax.experimental import pallas as pl
from jax.experimental.pallas import tpu as pltpu
from jax.experimental.pallas import tpu_sc as plsc

D_MODEL = 128
MAX_DEPTH = 300
N_IDX = 819200

NC = 2
NS = 16
NW = NC * NS
B_PER_W = N_IDX // NW
CHUNK = 128
N_CHUNKS = B_PER_W // CHUNK

_mesh = plsc.VectorSubcoreMesh(core_axis_name="c", subcore_axis_name="s")


@functools.partial(
    pl.kernel,
    out_type=jax.ShapeDtypeStruct((N_IDX, D_MODEL), jnp.float32),
    mesh=_mesh,
    scratch_types=[
        pltpu.VMEM((N_CHUNKS, CHUNK), jnp.int32),
        pltpu.VMEM((CHUNK, D_MODEL), jnp.float32),
        pltpu.SemaphoreType.DMA,
    ],
)
def _gather_kernel(x_hbm, pe_hbm, out_hbm, idx_v, rows_v, sem):
    wid = lax.axis_index("s") * NC + lax.axis_index("c")
    base = wid * B_PER_W
    pltpu.sync_copy(x_hbm.at[wid], idx_v)

    def body(i, carry):
        pltpu.async_copy(pe_hbm.at[idx_v.at[i]], rows_v, sem).wait()
        pltpu.sync_copy(rows_v, out_hbm.at[pl.ds(base + i * CHUNK, CHUNK)])
        return carry

    lax.fori_loop(0, N_CHUNKS, body, 0)


def kernel(x, pe):
    x3 = x.astype(jnp.int32).reshape(NW, N_CHUNKS, CHUNK)
    return _gather_kernel(x3, pe)


# speedup vs baseline: 3.9145x; 1.0084x over previous
import functools

import jax
import jax.numpy as jnp
from jax import lax
from jax.experimental import pallas as pl
from jax.experimental.pallas import tpu as pltpu
from jax.experimental.pallas import tpu_sc as plsc

D_MODEL = 128
MAX_DEPTH = 300
N_IDX = 819200

NC = 2
NS = 16
NW = NC * NS
B_PER_W = N_IDX // NW
CHUNK = 128
N_CHUNKS = B_PER_W // CHUNK

NBUF = 4
N_ROUNDS = N_CHUNKS // NBUF

_mesh = plsc.VectorSubcoreMesh(core_axis_name="c", subcore_axis_name="s")


@functools.partial(
    pl.kernel,
    out_type=jax.ShapeDtypeStruct((N_IDX, D_MODEL), jnp.float32),
    mesh=_mesh,
    scratch_types=[
        pltpu.VMEM((N_CHUNKS, CHUNK), jnp.int32),
        [pltpu.VMEM((CHUNK, D_MODEL), jnp.float32) for _ in range(NBUF)],
        [pltpu.SemaphoreType.DMA for _ in range(NBUF)],
        [pltpu.SemaphoreType.DMA for _ in range(NBUF)],
    ],
)
def _gather_kernel(x_hbm, pe_hbm, out_hbm, idx_v, rows, gsem, ssem):
    wid = lax.axis_index("s") * NC + lax.axis_index("c")
    base = wid * B_PER_W
    pltpu.sync_copy(x_hbm.at[wid], idx_v)

    def start_gather(i, b):
        pltpu.async_copy(pe_hbm.at[idx_v.at[i]], rows[b], gsem[b])

    def start_store(i, b):
        pltpu.async_copy(rows[b], out_hbm.at[pl.ds(base + i * CHUNK, CHUNK)],
                         ssem[b])

    def wait_gather(i, b):
        pltpu.make_async_copy(pe_hbm.at[idx_v.at[i]], rows[b], gsem[b]).wait()

    def wait_store(i, b):
        pltpu.make_async_copy(
            rows[b], out_hbm.at[pl.ds(base + i * CHUNK, CHUNK)], ssem[b]
        ).wait()

    for b in range(NBUF):
        start_gather(b, b)

    def round_body(r, carry):
        i0 = r * NBUF
        for b in range(NBUF):
            wait_gather(i0 + b, b)
            start_store(i0 + b, b)
        for b in range(NBUF):
            wait_store(i0 + b, b)
            start_gather(i0 + NBUF + b, b)
        return carry

    lax.fori_loop(0, N_ROUNDS - 1, round_body, 0)

    i0 = (N_ROUNDS - 1) * NBUF
    for b in range(NBUF):
        wait_gather(i0 + b, b)
        start_store(i0 + b, b)
    for b in range(NBUF):
        wait_store(i0 + b, b)


def kernel(x, pe):
    x3 = x.astype(jnp.int32).reshape(NW, N_CHUNKS, CHUNK)
    return _gather_kernel(x3, pe)


# speedup vs baseline: 17.0570x; 4.3574x over previous
import functools

import jax
import jax.numpy as jnp
from jax import lax
from jax.experimental import pallas as pl
from jax.experimental.pallas import tpu as pltpu
from jax.experimental.pallas import tpu_sc as plsc

D_MODEL = 128
MAX_DEPTH = 300
N_IDX = 819200

NC = 2
NS = 16
NW = NC * NS
B_PER_W = N_IDX // NW
CHUNK = 128
N_CHUNKS = B_PER_W // CHUNK

NBUF = 4
N_ROUNDS = N_CHUNKS // NBUF

_mesh = plsc.VectorSubcoreMesh(core_axis_name="c", subcore_axis_name="s")


@functools.partial(
    pl.kernel,
    out_type=jax.ShapeDtypeStruct((N_IDX, D_MODEL), jnp.float32),
    mesh=_mesh,
    scratch_types=[
        pltpu.VMEM((N_CHUNKS, CHUNK), jnp.int32),
        pltpu.VMEM_SHARED((MAX_DEPTH, D_MODEL), jnp.float32),
        [pltpu.VMEM((CHUNK, D_MODEL), jnp.float32) for _ in range(NBUF)],
        [pltpu.SemaphoreType.DMA for _ in range(NBUF)],
        [pltpu.SemaphoreType.DMA for _ in range(NBUF)],
    ],
)
def _gather_kernel(x_hbm, pe_hbm, out_hbm, idx_v, pe_sh, rows, gsem, ssem):
    wid = lax.axis_index("s") * NC + lax.axis_index("c")
    base = wid * B_PER_W
    @pl.when(lax.axis_index("s") == 0)
    def _():
        pltpu.sync_copy(pe_hbm, pe_sh)

    pltpu.sync_copy(x_hbm.at[wid], idx_v)
    plsc.subcore_barrier()

    def start_gather(i, b):
        pltpu.async_copy(pe_sh.at[idx_v.at[i]], rows[b], gsem[b])

    def start_store(i, b):
        pltpu.async_copy(rows[b], out_hbm.at[pl.ds(base + i * CHUNK, CHUNK)],
                         ssem[b])

    def wait_gather(i, b):
        pltpu.make_async_copy(pe_sh.at[idx_v.at[i]], rows[b], gsem[b]).wait()

    def wait_store(i, b):
        pltpu.make_async_copy(
            rows[b], out_hbm.at[pl.ds(base + i * CHUNK, CHUNK)], ssem[b]
        ).wait()

    for b in range(NBUF):
        start_gather(b, b)

    def round_body(r, carry):
        i0 = r * NBUF
        for b in range(NBUF):
            wait_gather(i0 + b, b)
            start_store(i0 + b, b)
        for b in range(NBUF):
            wait_store(i0 + b, b)
            start_gather(i0 + NBUF + b, b)
        return carry

    lax.fori_loop(0, N_ROUNDS - 1, round_body, 0)

    i0 = (N_ROUNDS - 1) * NBUF
    for b in range(NBUF):
        wait_gather(i0 + b, b)
        start_store(i0 + b, b)
    for b in range(NBUF):
        wait_store(i0 + b, b)


def kernel(x, pe):
    x3 = x.astype(jnp.int32).reshape(NW, N_CHUNKS, CHUNK)
    return _gather_kernel(x3, pe)
